# baseline (speedup 1.0000x reference)
import functools

import jax
import jax.numpy as jnp
from jax import lax
from jax.experimental import pallas as pl
from jax.experimental.pallas import tpu as pltpu
from jax.experimental.pallas import tpu_sc as plsc

H = 64
MAXF = 20
NTOK = 4096 * 50
NW = 32
TPW = NTOK // NW
C = 128
NCH = TPW // C


def _make_sc_embed(apply_tanh):
    mesh = plsc.VectorSubcoreMesh(core_axis_name="c", subcore_axis_name="s")

    @functools.partial(
        pl.kernel,
        out_type=jax.ShapeDtypeStruct((NTOK * H,), jnp.float32),
        mesh=mesh,
        scratch_types=[
            [pltpu.VMEM((3, C), jnp.int32)] * 2,
            [pltpu.VMEM((C,), jnp.int32)] * 2,
            [pltpu.VMEM((C,), jnp.int32)] * 2,
            [pltpu.VMEM((C, H), jnp.float32)] * 2,
            [pltpu.VMEM((C, H), jnp.float32)] * 2,
            [pltpu.VMEM((C * H,), jnp.float32)] * 2,
            [pltpu.SemaphoreType.DMA] * 2,
            [pltpu.SemaphoreType.DMA] * 2,
            [pltpu.SemaphoreType.DMA] * 2,
        ],
        compiler_params=pltpu.CompilerParams(use_tc_tiling_on_sc=False),
    )
    def body(stk_hbm, w_hbm, wc_hbm, h_hbm,
             sidx, tidx, cidx, wb, tb, ob, ssem, gsem, osem):
        wid = lax.axis_index("s") * 2 + lax.axis_index("c")
        base0 = wid * TPW

        def stage(g, s):
            pltpu.async_copy(
                stk_hbm.at[:, pl.ds(base0 + g * C, C)], sidx[s], ssem[s])

        def wait_stage(s):
            pltpu.make_async_copy(
                stk_hbm.at[:, pl.ds(0, C)], sidx[s], ssem[s]).wait()

        def fire(g, s):
            wait_stage(s)
            for i in range(C // 16):
                sl = pl.ds(i * 16, 16)
                tidx[s][sl] = sidx[s][0, sl]
                cidx[s][sl] = (jnp.minimum(sidx[s][1, sl], MAXF - 1) * MAXF
                               + jnp.minimum(sidx[s][2, sl], MAXF - 1))
            pltpu.async_copy(w_hbm.at[tidx[s]], wb[s], gsem[s])
            pltpu.async_copy(wc_hbm.at[cidx[s]], tb[s], gsem[s])

        def wait_gathers(s):
            pltpu.make_async_copy(w_hbm.at[tidx[s]], wb[s], gsem[s]).wait()
            pltpu.make_async_copy(wc_hbm.at[cidx[s]], tb[s], gsem[s]).wait()

        def compute(s):
            def row(r, carry):
                for j in range(H // 16):
                    sl = pl.ds(j * 16, 16)
                    fl = pl.ds(r * H + j * 16, 16)
                    x = wb[s][r, sl] + tb[s][r, sl]
                    if apply_tanh:
                        e = jnp.exp(x + x)
                        x = 1.0 - 2.0 / (e + 1.0)
                    ob[s][fl] = x
                return carry

            lax.fori_loop(0, C, row, None, unroll=False)

        def fire_outs(g, s):
            base = (base0 + g * C) * H
            pltpu.async_copy(ob[s], h_hbm.at[pl.ds(base, C * H)], osem[s])

        def drain_outs(s):
            pltpu.make_async_copy(
                ob[s], h_hbm.at[pl.ds(0, C * H)], osem[s]).wait()

        stage(0, 0)
        stage(1, 1)
        fire(0, 0)

        def step(gg, carry):
            g0 = 2 * gg
            wait_gathers(0)

            @pl.when(gg < NCH // 2 - 1)
            def _():
                stage(g0 + 2, 0)

            @pl.when(gg > 0)
            def _():
                drain_outs(1)

            fire(g0 + 1, 1)
            compute(0)
            fire_outs(g0, 0)

            wait_gathers(1)

            @pl.when(gg < NCH // 2 - 1)
            def _():
                stage(g0 + 3, 1)

            drain_outs(0)

            @pl.when(gg < NCH // 2 - 1)
            def _():
                fire(g0 + 2, 0)

            compute(1)
            fire_outs(g0 + 1, 1)
            return carry

        lax.fori_loop(0, NCH // 2, step, None, unroll=False)
        drain_outs(1)

    return body


_sc_embed_tanh = _make_sc_embed(True)
_sc_embed_plain = _make_sc_embed(False)


def _mask_body(tids_ref, ta_ref, tb_ref, bx_ref, sizes_ref, pm_ref):
    t = tids_ref[...]
    bx = t == 0
    bx_ref[...] = bx
    nz = jnp.logical_not(bx)
    sizes_ref[...] = jnp.sum(nz.astype(jnp.int32), axis=0, keepdims=True)
    L, BB = t.shape
    nza = ta_ref[...] != 0
    nzb = tb_ref[...] != 0
    pm_ref[...] = jnp.logical_and(
        jnp.broadcast_to(nza, (L, L, BB)),
        jnp.broadcast_to(nzb, (L, L, BB)))


def _tc_masks(doc_tids):
    B, L = doc_tids.shape
    BB = 1024
    tT = doc_tids.T
    bxT, sizesT, pmT = pl.pallas_call(
        _mask_body,
        grid=(B // BB,),
        in_specs=[
            pl.BlockSpec((L, BB), lambda i: (0, i)),
            pl.BlockSpec((L, 1, BB), lambda i: (0, 0, i)),
            pl.BlockSpec((1, L, BB), lambda i: (0, 0, i)),
        ],
        out_specs=(
            pl.BlockSpec((L, BB), lambda i: (0, i)),
            pl.BlockSpec((1, BB), lambda i: (0, i)),
            pl.BlockSpec((L, L, BB), lambda i: (0, 0, i)),
        ),
        out_shape=(
            jax.ShapeDtypeStruct((L, B), jnp.bool_),
            jax.ShapeDtypeStruct((1, B), jnp.int32),
            jax.ShapeDtypeStruct((L, L, B), jnp.bool_),
        ),
    )(tT, tT.reshape(L, 1, B), tT.reshape(1, L, B))
    return bxT.T, sizesT.T, jnp.transpose(pmT, (2, 0, 1))


def kernel(doc_tids, TFs, DFs, W_key, W_query, W_value, W_tf, W_df):
    B, L = doc_tids.shape
    tids = doc_tids.astype(jnp.int32)
    w_comb = (W_tf[:, None, :] + W_df[None, :, :]).reshape(MAXF * MAXF, H)
    bx_packed, doc_sizes, pad_mask = _tc_masks(tids)
    stk = jnp.stack([tids.reshape(-1),
                     TFs.astype(jnp.int32).reshape(-1),
                     DFs.astype(jnp.int32).reshape(-1)])
    hk = _sc_embed_tanh(stk, W_key, w_comb)
    hq = _sc_embed_tanh(stk, W_query, w_comb)
    hv = _sc_embed_plain(stk, W_value, w_comb)
    shape3 = (B, L, H)
    return (hk.reshape(shape3), hq.reshape(shape3), hv.reshape(shape3),
            bx_packed, doc_sizes, pad_mask)


# speedup vs baseline: 1.0812x; 1.0812x over previous
import functools

import jax
import jax.numpy as jnp
from jax import lax
from jax.experimental import pallas as pl
from jax.experimental.pallas import tpu as pltpu
from jax.experimental.pallas import tpu_sc as plsc

H = 64
MAXF = 20
NTOK = 4096 * 50
NW = 32
TPW = NTOK // NW
C = 128
NCH = TPW // C


def _sc_embed(stk, w_key, w_query, w_value, w_comb):
    mesh = plsc.VectorSubcoreMesh(core_axis_name="c", subcore_axis_name="s")

    @functools.partial(
        pl.kernel,
        out_type=(jax.ShapeDtypeStruct((NTOK, H), jnp.float32),) * 3,
        mesh=mesh,
        scratch_types=[
            [pltpu.VMEM((3, C), jnp.int32)] * 2,
            [pltpu.VMEM((C,), jnp.int32)] * 2,
            [pltpu.VMEM((C,), jnp.int32)] * 2,
            [pltpu.VMEM((C, H), jnp.float32)] * 2,
            [pltpu.VMEM((C, H), jnp.float32)] * 2,
            [pltpu.VMEM((C, H), jnp.float32)] * 2,
            [pltpu.VMEM((C, H), jnp.float32)] * 2,
            [pltpu.VMEM((C, H), jnp.float32)] * 2,
            [pltpu.VMEM((C, H), jnp.float32)] * 2,
            [pltpu.VMEM((C, H), jnp.float32)] * 2,
            [pltpu.SemaphoreType.DMA] * 2,
            [pltpu.SemaphoreType.DMA] * 2,
            [pltpu.SemaphoreType.DMA] * 2,
        ],
        compiler_params=pltpu.CompilerParams(use_tc_tiling_on_sc=False),
    )
    def body(stk_hbm, wk_hbm, wq_hbm, wv_hbm, wc_hbm,
             hk_hbm, hq_hbm, hv_hbm,
             sidx, tidx, cidx, kb, qb, vb, tb, ko, qo, vo, ssem, gsem, osem):
        wid = lax.axis_index("s") * 2 + lax.axis_index("c")
        base0 = wid * TPW

        def stage(g, s):
            pltpu.async_copy(
                stk_hbm.at[:, pl.ds(base0 + g * C, C)], sidx[s], ssem[s])

        def wait_stage(s):
            pltpu.make_async_copy(
                stk_hbm.at[:, pl.ds(0, C)], sidx[s], ssem[s]).wait()

        def fire(g, s):
            wait_stage(s)
            for i in range(C // 16):
                sl = pl.ds(i * 16, 16)
                tidx[s][sl] = sidx[s][0, sl]
                cidx[s][sl] = (jnp.minimum(sidx[s][1, sl], MAXF - 1) * MAXF
                               + jnp.minimum(sidx[s][2, sl], MAXF - 1))
            pltpu.async_copy(wk_hbm.at[tidx[s]], kb[s], gsem[s])
            pltpu.async_copy(wq_hbm.at[tidx[s]], qb[s], gsem[s])
            pltpu.async_copy(wv_hbm.at[tidx[s]], vb[s], gsem[s])
            pltpu.async_copy(wc_hbm.at[cidx[s]], tb[s], gsem[s])

        def wait_gathers(s):
            pltpu.make_async_copy(wk_hbm.at[tidx[s]], kb[s], gsem[s]).wait()
            pltpu.make_async_copy(wq_hbm.at[tidx[s]], qb[s], gsem[s]).wait()
            pltpu.make_async_copy(wv_hbm.at[tidx[s]], vb[s], gsem[s]).wait()
            pltpu.make_async_copy(wc_hbm.at[cidx[s]], tb[s], gsem[s]).wait()

        def compute(s):
            def row(r, carry):
                for j in range(H // 16):
                    sl = pl.ds(j * 16, 16)
                    t = tb[s][r, sl]
                    k = kb[s][r, sl] + t
                    q = qb[s][r, sl] + t
                    vo[s][r, sl] = vb[s][r, sl] + t
                    ek = jnp.exp(k + k)
                    ko[s][r, sl] = 1.0 - 2.0 / (ek + 1.0)
                    eq = jnp.exp(q + q)
                    qo[s][r, sl] = 1.0 - 2.0 / (eq + 1.0)
                return carry

            lax.fori_loop(0, C, row, None, unroll=False)

        def fire_outs(g, s):
            base = base0 + g * C
            pltpu.async_copy(ko[s], hk_hbm.at[pl.ds(base, C)], osem[s])
            pltpu.async_copy(qo[s], hq_hbm.at[pl.ds(base, C)], osem[s])
            pltpu.async_copy(vo[s], hv_hbm.at[pl.ds(base, C)], osem[s])

        def drain_outs(s):
            pltpu.make_async_copy(
                ko[s], hk_hbm.at[pl.ds(0, C)], osem[s]).wait()
            pltpu.make_async_copy(
                qo[s], hq_hbm.at[pl.ds(0, C)], osem[s]).wait()
            pltpu.make_async_copy(
                vo[s], hv_hbm.at[pl.ds(0, C)], osem[s]).wait()

        stage(0, 0)
        stage(1, 1)
        fire(0, 0)

        def step(gg, carry):
            g0 = 2 * gg
            wait_gathers(0)

            @pl.when(gg < NCH // 2 - 1)
            def _():
                stage(g0 + 2, 0)

            @pl.when(gg > 0)
            def _():
                drain_outs(1)

            fire(g0 + 1, 1)
            compute(0)
            fire_outs(g0, 0)
            wait_gathers(1)

            @pl.when(gg < NCH // 2 - 1)
            def _():
                stage(g0 + 3, 1)

            drain_outs(0)

            @pl.when(gg < NCH // 2 - 1)
            def _():
                fire(g0 + 2, 0)

            compute(1)
            fire_outs(g0 + 1, 1)
            return carry

        lax.fori_loop(0, NCH // 2, step, None, unroll=False)
        drain_outs(1)

    return body(stk, w_key, w_query, w_value, w_comb)


def _mask_body(tids_ref, ta_ref, tb_ref, bx_ref, sizes_ref, pm_ref):
    t = tids_ref[...]
    bx = t == 0
    bx_ref[...] = bx
    nz = jnp.logical_not(bx)
    sizes_ref[...] = jnp.sum(nz.astype(jnp.int32), axis=0, keepdims=True)
    L, BB = t.shape
    nza = ta_ref[...] != 0
    nzb = tb_ref[...] != 0
    pm_ref[...] = jnp.logical_and(
        jnp.broadcast_to(nza, (L, L, BB)),
        jnp.broadcast_to(nzb, (L, L, BB)))


def _tc_masks(doc_tids):
    B, L = doc_tids.shape
    BB = 1024
    tT = doc_tids.T
    bxT, sizesT, pmT = pl.pallas_call(
        _mask_body,
        grid=(B // BB,),
        in_specs=[
            pl.BlockSpec((L, BB), lambda i: (0, i)),
            pl.BlockSpec((L, 1, BB), lambda i: (0, 0, i)),
            pl.BlockSpec((1, L, BB), lambda i: (0, 0, i)),
        ],
        out_specs=(
            pl.BlockSpec((L, BB), lambda i: (0, i)),
            pl.BlockSpec((1, BB), lambda i: (0, i)),
            pl.BlockSpec((L, L, BB), lambda i: (0, 0, i)),
        ),
        out_shape=(
            jax.ShapeDtypeStruct((L, B), jnp.bool_),
            jax.ShapeDtypeStruct((1, B), jnp.int32),
            jax.ShapeDtypeStruct((L, L, B), jnp.bool_),
        ),
    )(tT, tT.reshape(L, 1, B), tT.reshape(1, L, B))
    return bxT.T, sizesT.T, jnp.transpose(pmT, (2, 0, 1))


def kernel(doc_tids, TFs, DFs, W_key, W_query, W_value, W_tf, W_df):
    B, L = doc_tids.shape
    tids = doc_tids.astype(jnp.int32)
    w_comb = (W_tf[:, None, :] + W_df[None, :, :]).reshape(MAXF * MAXF, H)
    bx_packed, doc_sizes, pad_mask = _tc_masks(tids)
    stk = jnp.stack([tids.reshape(-1),
                     TFs.astype(jnp.int32).reshape(-1),
                     DFs.astype(jnp.int32).reshape(-1)])
    hk, hq, hv = _sc_embed(stk, W_key, W_query, W_value, w_comb)
    shape3 = (B, L, H)
    return (hk.reshape(shape3), hq.reshape(shape3), hv.reshape(shape3),
            bx_packed, doc_sizes, pad_mask)
